# baseline (device time: 232765 ns/iter reference)
import jax
import jax.numpy as jnp
from jax import lax
from jax.experimental import pallas as pl
from jax.experimental.pallas import tpu as pltpu

N_DEV = 4
N_TOK = 4096
D_MODEL = 1024
H = 2048
HB = H // 2
E_LOCAL = 4
CHUNK = N_TOK // N_DEV
N_STEPS = 2 * (N_DEV - 1)


def kernel(x, router_W, route_idx, expert_W):
    del router_W
    xb = x.astype(jnp.bfloat16)
    wb = expert_W.astype(jnp.bfloat16).reshape(E_LOCAL * D_MODEL, H)

    def body(x_ref, idx_ref, w_ref, out_ref, comm_cw, comm_ccw, xm_ref,
             acc_cw, acc_ccw, send_cw, recv_cw, send_ccw, recv_ccw, copy_sems):
        my = lax.axis_index("i")
        left = lax.rem(my + N_DEV - 1, N_DEV)
        right = lax.rem(my + 1, N_DEV)

        barrier_sem = pltpu.get_barrier_semaphore()
        for nbr in (left, right):
            pl.semaphore_signal(
                barrier_sem, inc=1,
                device_id=(nbr,), device_id_type=pl.DeviceIdType.MESH,
            )
        pl.semaphore_wait(barrier_sem, 2)

        def build_xcat(c):
            xc = x_ref[pl.ds(c * CHUNK, CHUNK), :]
            ic = idx_ref[pl.ds(c * CHUNK, CHUNK), :]
            for le in range(E_LOCAL):
                xm_ref[:, pl.ds(le * D_MODEL, D_MODEL)] = jnp.where(
                    ic == my * E_LOCAL + le, xc, jnp.zeros_like(xc)
                )

        def matmul_half(dst, half):
            dst[...] = jnp.dot(
                xm_ref[...],
                w_ref[:, pl.ds(half * HB, HB)],
                preferred_element_type=jnp.float32,
            ).astype(jnp.bfloat16)

        def partial_pair(c_cw, c_ccw, shared, dst_cw, dst_ccw):
            build_xcat(c_cw)
            matmul_half(dst_cw, 0)
            if not shared:
                build_xcat(c_ccw)
            matmul_half(dst_ccw, 1)

        def store_half(c, src, half, sem_idx):
            cp = pltpu.make_async_copy(
                src,
                out_ref.at[pl.ds(c * CHUNK, CHUNK), pl.ds(half * HB, HB)],
                copy_sems.at[sem_idx],
            )
            cp.start()
            return cp

        partial_pair(my, my, True, comm_cw.at[0], comm_ccw.at[0])

        for s in range(N_STEPS):
            send_slot = s % 3
            recv_slot = (s + 1) % 3
            rdma_cw = pltpu.make_async_remote_copy(
                src_ref=comm_cw.at[send_slot],
                dst_ref=comm_cw.at[recv_slot],
                send_sem=send_cw.at[s],
                recv_sem=recv_cw.at[s],
                device_id=(right,),
                device_id_type=pl.DeviceIdType.MESH,
            )
            rdma_ccw = pltpu.make_async_remote_copy(
                src_ref=comm_ccw.at[send_slot],
                dst_ref=comm_ccw.at[recv_slot],
                send_sem=send_ccw.at[s],
                recv_sem=recv_ccw.at[s],
                device_id=(left,),
                device_id_type=pl.DeviceIdType.MESH,
            )
            rdma_cw.start()
            rdma_ccw.start()

            if s < N_DEV - 1:
                cr_cw = lax.rem(my - 1 - s + 2 * N_DEV, N_DEV)
                cr_ccw = lax.rem(my + 1 + s, N_DEV)
                partial_pair(cr_cw, cr_ccw, s % 2 == 1, acc_cw, acc_ccw)
                rdma_cw.wait()
                rdma_ccw.wait()
                comm_cw[recv_slot, :, :] += acc_cw[...]
                comm_ccw[recv_slot, :, :] += acc_ccw[...]
                if s == N_DEV - 2:
                    cp0 = store_half(cr_cw, comm_cw.at[recv_slot], 0, 0)
                    cp1 = store_half(cr_ccw, comm_ccw.at[recv_slot], 1, 1)
                    cp0.wait()
                    cp1.wait()
            else:
                t = s - (N_DEV - 1)
                cr_cw = lax.rem(my - t + N_DEV, N_DEV)
                cr_ccw = lax.rem(my + t, N_DEV)
                rdma_cw.wait()
                rdma_ccw.wait()
                cp0 = store_half(cr_cw, comm_cw.at[recv_slot], 0, 0)
                cp1 = store_half(cr_ccw, comm_ccw.at[recv_slot], 1, 1)
                cp0.wait()
                cp1.wait()

    out = pl.pallas_call(
        body,
        out_shape=jax.ShapeDtypeStruct((N_TOK, H), jnp.bfloat16),
        in_specs=[
            pl.BlockSpec(memory_space=pltpu.VMEM),
            pl.BlockSpec(memory_space=pltpu.VMEM),
            pl.BlockSpec(memory_space=pltpu.VMEM),
        ],
        out_specs=pl.BlockSpec(memory_space=pl.ANY),
        scratch_shapes=[
            pltpu.VMEM((3, CHUNK, HB), jnp.bfloat16),
            pltpu.VMEM((3, CHUNK, HB), jnp.bfloat16),
            pltpu.VMEM((CHUNK, E_LOCAL * D_MODEL), jnp.bfloat16),
            pltpu.VMEM((CHUNK, HB), jnp.bfloat16),
            pltpu.VMEM((CHUNK, HB), jnp.bfloat16),
            pltpu.SemaphoreType.DMA((N_STEPS,)),
            pltpu.SemaphoreType.DMA((N_STEPS,)),
            pltpu.SemaphoreType.DMA((N_STEPS,)),
            pltpu.SemaphoreType.DMA((N_STEPS,)),
            pltpu.SemaphoreType.DMA((2,)),
        ],
        compiler_params=pltpu.CompilerParams(
            collective_id=0, vmem_limit_bytes=100 * 1024 * 1024
        ),
    )(xb, route_idx, wb)
    return out.astype(jnp.float32)
